# baseline (speedup 1.0000x reference)
import functools

import jax
import jax.numpy as jnp
from jax import lax
from jax.experimental import pallas as pl
from jax.experimental.pallas import tpu as pltpu
from jax.experimental.pallas import tpu_sc as plsc

N = 16384
C = 1000
NW = 32
RPW = N // NW
CH = 16
NCH = RPW // CH
CFULL = (C // 16) * 16
CTAIL = C - CFULL


def _sc_body(idx_hbm, out_hbm, idx_v, buf0, buf1, sem):
    wid = lax.axis_index("s") * 2 + lax.axis_index("c")
    base_row = wid * RPW
    pltpu.sync_copy(idx_hbm.at[pl.ds(base_row, RPW)], idx_v)

    lanes = lax.iota(jnp.int32, 16)
    zeros = jnp.zeros((16,), jnp.float32)
    ones = jnp.ones((16,), jnp.float32)
    tail_mask = lanes < CTAIL

    for buf in (buf0, buf1):
        def _zero_row(r, _, buf=buf):
            for cc in range(CFULL // 16):
                buf[r, pl.ds(cc * 16, 16)] = zeros
            rows = jnp.full((16,), r, jnp.int32)
            plsc.store_scatter(buf, [rows, CFULL + lanes], zeros,
                               mask=tail_mask)
            return _
        lax.fori_loop(0, CH, _zero_row, 0)

    bufs = (buf0, buf1)
    copies = [None] * NCH
    for c in range(NCH):
        b = bufs[c % 2]
        cols = idx_v[pl.ds(c * CH, 16)]
        if c >= 2:
            copies[c - 2].wait()
            oldcols = idx_v[pl.ds((c - 2) * CH, 16)]
            plsc.store_scatter(b, [lanes, oldcols], zeros)
        plsc.store_scatter(b, [lanes, cols], ones)
        copies[c] = pltpu.async_copy(
            b, out_hbm.at[pl.ds(base_row + c * CH, CH)], sem.at[c % 2])
    copies[NCH - 2].wait()
    copies[NCH - 1].wait()


def kernel(inputs):
    idx = inputs.astype(jnp.int32)
    mesh = plsc.VectorSubcoreMesh(core_axis_name="c", subcore_axis_name="s")
    run = functools.partial(
        pl.kernel,
        mesh=mesh,
        out_type=jax.ShapeDtypeStruct((N, C), jnp.float32),
        scratch_types=[
            pltpu.VMEM((RPW,), jnp.int32),
            pltpu.VMEM((CH, C), jnp.float32),
            pltpu.VMEM((CH, C), jnp.float32),
            pltpu.SemaphoreType.DMA((2,)),
        ],
        compiler_params=pltpu.CompilerParams(needs_layout_passes=False),
    )(_sc_body)
    return run(idx)
